# Initial kernel scaffold; baseline (speedup 1.0000x reference)
import jax
import jax.numpy as jnp
from jax.experimental import pallas as pl


def kernel(batchsize, input):
    raise NotImplementedError("write your pallas kernel here")


# speedup vs baseline: 33.0426x; 33.0426x over previous
import functools

import jax
import jax.numpy as jnp
import numpy as np
from jax import lax
from jax.experimental import pallas as pl
from jax.experimental.pallas import tpu as pltpu
from jax.experimental.pallas import tpu_sc as plsc

NC, NS, L = 2, 16, 16
NW = NC * NS
NBINS = 256
NCH = 3
IMG = 512 * 512
IMG3 = NCH * IMG
CH = 16384
CHUNKS = IMG3 // CH
CH_PER_PLANE = IMG // CH
GROUPS = CH // L
UNROLL = 8
ACC = NCH * NBINS * L
PROW = NCH * NBINS
SCALE = np.float32(256.0 / 255.0)


def _sc_hist(in_hbm, out_hbm, buf, acc, orow, sem0, sem1):
    wid = lax.axis_index("s") * NC + lax.axis_index("c")
    base = wid * IMG3
    sems = (sem0, sem1)
    lane = jax.lax.iota(jnp.int32, 16)
    ones = jnp.full((16,), 1.0, jnp.float32)

    def copy(ch, b):
        return pltpu.make_async_copy(
            in_hbm.at[pl.ds(base + ch * CH, CH)], buf.at[b], sems[b])

    def zbody(j, _):
        for u in range(UNROLL):
            acc[pl.ds((j * UNROLL + u) * L, L)] = jnp.zeros((16,), jnp.float32)
        return 0
    lax.fori_loop(0, ACC // L // UNROLL, zbody, 0)

    copy(0, 0).start()
    copy(1, 1).start()

    def process(b, coff):
        bvec = lane + coff

        def gbody(j, _):
            for u in range(UNROLL):
                v = buf[b, pl.ds((j * UNROLL + u) * L, L)]
                y = (v * jnp.float32(255.0)) * SCALE
                idx = y.astype(jnp.int32)
                idx = lax.max(lax.min(idx, 255), 0)
                addr = idx * L + bvec
                plsc.addupdate_scatter(acc, (addr,), ones)
            return 0
        lax.fori_loop(0, GROUPS // UNROLL, gbody, 0)

    def ring(t, last):
        for b in range(2):
            ch = t * 2 + b
            copy(ch, b).wait()
            coff = (ch // CH_PER_PLANE) * (NBINS * L)
            process(b, coff)
            if not last:
                copy(ch + 2, b).start()

    lax.fori_loop(0, CHUNKS // 2 - 1, lambda t, _: (ring(t, False), 0)[1], 0)
    ring(CHUNKS // 2 - 1, True)

    def rbody(j, _):
        rbase = j * L * L + lane * L
        s = plsc.load_gather(acc, (rbase,))
        for l in range(1, L):
            s = s + plsc.load_gather(acc, (rbase + l,))
        orow[pl.ds(j * L, L)] = s
        return 0
    lax.fori_loop(0, PROW // L, rbody, 0)

    pltpu.sync_copy(orow, out_hbm.at[wid])


@functools.partial(jax.jit, static_argnums=())
def _sc_partials(flat):
    mesh = plsc.VectorSubcoreMesh(core_axis_name="c", subcore_axis_name="s")
    f = pl.kernel(
        _sc_hist,
        out_type=jax.ShapeDtypeStruct((NW, PROW), jnp.float32),
        mesh=mesh,
        scratch_types=[
            pltpu.VMEM((2, CH), jnp.float32),
            pltpu.VMEM((ACC,), jnp.float32),
            pltpu.VMEM((PROW,), jnp.float32),
            pltpu.SemaphoreType.DMA,
            pltpu.SemaphoreType.DMA,
        ],
        compiler_params=pltpu.CompilerParams(needs_layout_passes=False),
    )
    return f(flat)


def _tc_reduce_body(p_ref, o_ref):
    o_ref[...] = jnp.sum(p_ref[...], axis=0, keepdims=True)


def kernel(batchsize, input):
    flat = jnp.reshape(input, (-1,))
    partials = _sc_partials(flat)
    sums = pl.pallas_call(
        _tc_reduce_body,
        out_shape=jax.ShapeDtypeStruct((1, PROW), jnp.float32),
    )(partials)[0]
    count_r = sums[:NBINS]
    count_g = sums[NBINS:2 * NBINS]
    count_b = sums[2 * NBINS:]
    hist_r_counts = partials[batchsize - 1, :NBINS]
    bins = jnp.linspace(0.0, 255.0, 257)
    return ((hist_r_counts, bins), count_r, count_g, count_b)
